# Initial kernel scaffold; baseline (speedup 1.0000x reference)
import jax
import jax.numpy as jnp
from jax.experimental import pallas as pl


def kernel(x, edge_index, edge_attr, batch, W_in, b_in, We, eps, W1, b1, W2, b2, gamma, beta, Wp, bp):
    raise NotImplementedError("write your pallas kernel here")


# speedup vs baseline: 1.1875x; 1.1875x over previous
import functools

import jax
import jax.numpy as jnp
from jax import lax
from jax.experimental import pallas as pl
from jax.experimental.pallas import tpu as pltpu
from jax.experimental.pallas import tpu_sc as plsc

_NC = 2
_NS = 16
_NW = _NC * _NS
_C = 128


def _sc_edge_agg(h, src, dst, attr_flat, we, ztile, n_table, chunks_per_worker):
    N, D = h.shape
    rows_per_tile = n_table // _NS
    nj = D // 16

    mesh = plsc.VectorSubcoreMesh(core_axis_name="c", subcore_axis_name="s")

    @functools.partial(
        pl.kernel,
        out_type=jax.ShapeDtypeStruct((_NC, n_table, D), jnp.float32),
        mesh=mesh,
        scratch_types=[
            pltpu.VMEM((_C,), jnp.int32),
            pltpu.VMEM((_C,), jnp.int32),
            pltpu.VMEM((_C * 4 + 16,), jnp.float32),
            pltpu.VMEM((_C, D), jnp.float32),
            pltpu.VMEM((_C, D), jnp.float32),
            pltpu.VMEM((4, D), jnp.float32),
            pltpu.VMEM_SHARED((n_table, D), jnp.float32),
            pltpu.SemaphoreType.DMA,
        ],
    )
    def k(h_hbm, src_hbm, dst_hbm, attr_hbm, we_hbm, z_hbm, out_hbm,
          src_v, dst_v, attr_v, rows_v, msg_v, we_v, agg_sh, sem):
        cid = lax.axis_index("c")
        sid = lax.axis_index("s")
        wid = sid * _NC + cid

        pltpu.sync_copy(z_hbm, agg_sh.at[pl.ds(sid * rows_per_tile, rows_per_tile)])
        pltpu.sync_copy(we_hbm, we_v)
        plsc.subcore_barrier()

        base0 = wid * (chunks_per_worker * _C)

        def chunk_body(ci, carry):
            base = pl.multiple_of(base0 + ci * _C, _C)
            pltpu.sync_copy(src_hbm.at[pl.ds(base, _C)], src_v)
            pltpu.sync_copy(dst_hbm.at[pl.ds(base, _C)], dst_v)
            pltpu.sync_copy(attr_hbm.at[pl.ds(base * 4, _C * 4)],
                            attr_v.at[pl.ds(0, _C * 4)])
            pltpu.async_copy(h_hbm.at[src_v], rows_v, sem).wait()

            def edge_body(i, c2):
                av = attr_v[pl.ds(4 * i, 16)]
                a0 = av[0]
                a1 = av[1]
                a2 = av[2]
                a3 = av[3]
                for j in range(nj):
                    sl = pl.ds(16 * j, 16)
                    e = (a0 * we_v[0, sl] + a1 * we_v[1, sl]
                         + a2 * we_v[2, sl] + a3 * we_v[3, sl])
                    msg_v[i, sl] = jnp.maximum(rows_v[i, sl] + e, 0.0)
                return c2

            lax.fori_loop(0, _C, edge_body, 0)
            pltpu.sync_copy(msg_v, agg_sh.at[dst_v], add=True)
            return carry

        lax.fori_loop(0, chunks_per_worker, chunk_body, 0)
        plsc.subcore_barrier()
        pltpu.sync_copy(
            agg_sh.at[pl.ds(sid * rows_per_tile, rows_per_tile)],
            out_hbm.at[cid, pl.ds(sid * rows_per_tile, rows_per_tile)],
        )

    return k(h, src, dst, attr_flat, we, ztile)


def _tc_encode(x, W_in, b_in, block_rows):
    N, D = x.shape

    def body(x_ref, w_ref, b_ref, o_ref):
        o_ref[...] = (
            jnp.dot(x_ref[...], w_ref[...], preferred_element_type=jnp.float32)
            + b_ref[...]
        )

    return pl.pallas_call(
        body,
        grid=(N // block_rows,),
        in_specs=[
            pl.BlockSpec((block_rows, D), lambda i: (i, 0)),
            pl.BlockSpec((D, D), lambda i: (0, 0)),
            pl.BlockSpec((1, D), lambda i: (0, 0)),
        ],
        out_specs=pl.BlockSpec((block_rows, D), lambda i: (i, 0)),
        out_shape=jax.ShapeDtypeStruct((N, D), jnp.float32),
    )(x, W_in, b_in.reshape(1, D))


def _tc_mlp(h, agg, w1, b1, w2, b2, gamma, beta, scal, relu_out, block_rows):
    N, D = h.shape
    H = w1.shape[1]
    n_table = agg.shape[1]

    def body(s_ref, h_ref, a_ref, w1_ref, b1_ref, w2_ref, b2_ref, g_ref, be_ref, o_ref):
        z = s_ref[0, 0] * h_ref[...] + a_ref[0] + a_ref[1]
        hid = jnp.maximum(
            jnp.dot(z, w1_ref[...], preferred_element_type=jnp.float32) + b1_ref[...],
            0.0,
        )
        o = (
            jnp.dot(hid, w2_ref[...], preferred_element_type=jnp.float32) + b2_ref[...]
        ) * g_ref[...] + be_ref[...]
        if relu_out:
            o = jnp.maximum(o, 0.0)
        o_ref[...] = o

    return pl.pallas_call(
        body,
        grid=(N // block_rows,),
        in_specs=[
            pl.BlockSpec(memory_space=pltpu.SMEM),
            pl.BlockSpec((block_rows, D), lambda i: (i, 0)),
            pl.BlockSpec((2, block_rows, D), lambda i: (0, i, 0)),
            pl.BlockSpec((D, H), lambda i: (0, 0)),
            pl.BlockSpec((1, H), lambda i: (0, 0)),
            pl.BlockSpec((H, D), lambda i: (0, 0)),
            pl.BlockSpec((1, D), lambda i: (0, 0)),
            pl.BlockSpec((1, D), lambda i: (0, 0)),
            pl.BlockSpec((1, D), lambda i: (0, 0)),
        ],
        out_specs=pl.BlockSpec((block_rows, D), lambda i: (i, 0)),
        out_shape=jax.ShapeDtypeStruct((N, D), jnp.float32),
    )(
        scal.reshape(1, 1), h, agg,
        w1, b1.reshape(1, H), w2, b2.reshape(1, D),
        gamma.reshape(1, D), beta.reshape(1, D),
    )


def _tc_pool(h, batch, Wp, bp, G):
    N, D = h.shape
    T = Wp.shape[1]

    def body(h_ref, b_ref, wp_ref, bp_ref, o_ref):
        gid = lax.broadcasted_iota(jnp.int32, (G, N), 0)
        pt = jnp.where(b_ref[...] == gid, 1.0, 0.0)
        ssum = jnp.dot(pt, h_ref[...], preferred_element_type=jnp.float32)
        cnt = jnp.dot(pt, jnp.ones((N, 1), jnp.float32),
                      preferred_element_type=jnp.float32)
        pooled = ssum / jnp.maximum(cnt, 1.0)
        o_ref[...] = (
            jnp.dot(pooled, wp_ref[...], preferred_element_type=jnp.float32)
            + bp_ref[...]
        )

    return pl.pallas_call(
        body,
        in_specs=[
            pl.BlockSpec((N, D), lambda: (0, 0)),
            pl.BlockSpec((1, N), lambda: (0, 0)),
            pl.BlockSpec((D, T), lambda: (0, 0)),
            pl.BlockSpec((1, T), lambda: (0, 0)),
        ],
        out_specs=pl.BlockSpec((G, T), lambda: (0, 0)),
        out_shape=jax.ShapeDtypeStruct((G, T), jnp.float32),
    )(h, batch.reshape(1, N), Wp, bp.reshape(1, T))


def kernel(x, edge_index, edge_attr, batch, W_in, b_in, We, eps, W1, b1, W2,
           b2, gamma, beta, Wp, bp):
    N, D = x.shape
    E = edge_index.shape[1]
    L = We.shape[0]
    G = 128
    block_rows = 1000

    chunks_per_worker = -(-E // (_NW * _C))
    e_pad = _NW * chunks_per_worker * _C
    n_table = -(-(N + 1) // (_NS * 8)) * (_NS * 8)

    src = jnp.concatenate([edge_index[0], jnp.zeros((e_pad - E,), jnp.int32)])
    dst = jnp.concatenate(
        [edge_index[1], jnp.full((e_pad - E,), N, jnp.int32)])
    attr_flat = jnp.concatenate(
        [edge_attr.reshape(-1), jnp.zeros(((e_pad - E) * 4,), jnp.float32)])
    ztile = jnp.zeros((n_table // _NS, D), jnp.float32)

    h = _tc_encode(x, W_in, b_in, block_rows)
    for l in range(L):
        agg = _sc_edge_agg(h, src, dst, attr_flat, We[l], ztile,
                           n_table, chunks_per_worker)
        h = _tc_mlp(h, agg, W1[l], b1[l], W2[l], b2[l], gamma[l], beta[l],
                    1.0 + eps[l], relu_out=(l < L - 1), block_rows=block_rows)
    return _tc_pool(h, batch, Wp, bp, G)


# speedup vs baseline: 3.1470x; 2.6501x over previous
import functools

import jax
import jax.numpy as jnp
from jax import lax
from jax.experimental import pallas as pl
from jax.experimental.pallas import tpu as pltpu
from jax.experimental.pallas import tpu_sc as plsc

_NC = 2
_NS = 16
_NW = _NC * _NS
_C = 128


def _sc_edge_agg(h, src, dst, attr_flat, we, ztile, n_table, chunks_per_worker):
    N, D = h.shape
    rows_per_tile = n_table // _NS
    nj = D // 16

    mesh = plsc.VectorSubcoreMesh(core_axis_name="c", subcore_axis_name="s")

    @functools.partial(
        pl.kernel,
        out_type=jax.ShapeDtypeStruct((_NC, n_table, D), jnp.float32),
        mesh=mesh,
        scratch_types=[
            pltpu.VMEM((_C,), jnp.int32),
            pltpu.VMEM((_C,), jnp.int32),
            pltpu.VMEM((_C * 4 + 16,), jnp.float32),
            pltpu.VMEM((_C, D), jnp.float32),
            pltpu.VMEM((_C, D), jnp.float32),
            pltpu.VMEM((4, D), jnp.float32),
            pltpu.VMEM_SHARED((n_table, D), jnp.float32),
            pltpu.SemaphoreType.DMA,
        ],
    )
    def k(h_hbm, src_hbm, dst_hbm, attr_hbm, we_hbm, z_hbm, out_hbm,
          src_v, dst_v, attr_v, rows_v, msg_v, we_v, agg_sh, sem):
        cid = lax.axis_index("c")
        sid = lax.axis_index("s")
        wid = sid * _NC + cid

        pltpu.sync_copy(z_hbm, agg_sh.at[pl.ds(sid * rows_per_tile, rows_per_tile)])
        pltpu.sync_copy(we_hbm, we_v)
        plsc.subcore_barrier()

        base0 = wid * (chunks_per_worker * _C)

        wreg = [[we_v[kk, pl.ds(16 * j, 16)] for kk in range(4)]
                for j in range(nj)]

        def chunk_body(ci, carry):
            base = pl.multiple_of(base0 + ci * _C, _C)
            pltpu.sync_copy(src_hbm.at[pl.ds(base, _C)], src_v)
            pltpu.sync_copy(dst_hbm.at[pl.ds(base, _C)], dst_v)
            pltpu.sync_copy(attr_hbm.at[pl.ds(base * 4, _C * 4)],
                            attr_v.at[pl.ds(0, _C * 4)])
            pltpu.async_copy(h_hbm.at[src_v], rows_v, sem).wait()

            @functools.partial(plsc.parallel_loop, 0, _C, unroll=2)
            def edge_body(i):
                av = attr_v[pl.ds(4 * i, 16)]
                a0 = av[0]
                a1 = av[1]
                a2 = av[2]
                a3 = av[3]
                for j in range(nj):
                    sl = pl.ds(16 * j, 16)
                    w0, w1, w2, w3 = wreg[j]
                    t0 = a0 * w0 + a1 * w1
                    t1 = a2 * w2 + a3 * w3
                    msg_v[i, sl] = jnp.maximum((rows_v[i, sl] + t0) + t1, 0.0)

            pltpu.sync_copy(msg_v, agg_sh.at[dst_v], add=True)
            return carry

        lax.fori_loop(0, chunks_per_worker, chunk_body, 0)
        plsc.subcore_barrier()
        pltpu.sync_copy(
            agg_sh.at[pl.ds(sid * rows_per_tile, rows_per_tile)],
            out_hbm.at[cid, pl.ds(sid * rows_per_tile, rows_per_tile)],
        )

    return k(h, src, dst, attr_flat, we, ztile)


def _tc_encode(x, W_in, b_in, block_rows):
    N, D = x.shape

    def body(x_ref, w_ref, b_ref, o_ref):
        o_ref[...] = (
            jnp.dot(x_ref[...], w_ref[...], preferred_element_type=jnp.float32)
            + b_ref[...]
        )

    return pl.pallas_call(
        body,
        grid=(N // block_rows,),
        in_specs=[
            pl.BlockSpec((block_rows, D), lambda i: (i, 0)),
            pl.BlockSpec((D, D), lambda i: (0, 0)),
            pl.BlockSpec((1, D), lambda i: (0, 0)),
        ],
        out_specs=pl.BlockSpec((block_rows, D), lambda i: (i, 0)),
        out_shape=jax.ShapeDtypeStruct((N, D), jnp.float32),
    )(x, W_in, b_in.reshape(1, D))


def _tc_mlp(h, agg, w1, b1, w2, b2, gamma, beta, scal, relu_out, block_rows):
    N, D = h.shape
    H = w1.shape[1]
    n_table = agg.shape[1]

    def body(s_ref, h_ref, a_ref, w1_ref, b1_ref, w2_ref, b2_ref, g_ref, be_ref, o_ref):
        z = s_ref[0, 0] * h_ref[...] + a_ref[0] + a_ref[1]
        hid = jnp.maximum(
            jnp.dot(z, w1_ref[...], preferred_element_type=jnp.float32) + b1_ref[...],
            0.0,
        )
        o = (
            jnp.dot(hid, w2_ref[...], preferred_element_type=jnp.float32) + b2_ref[...]
        ) * g_ref[...] + be_ref[...]
        if relu_out:
            o = jnp.maximum(o, 0.0)
        o_ref[...] = o

    return pl.pallas_call(
        body,
        grid=(N // block_rows,),
        in_specs=[
            pl.BlockSpec(memory_space=pltpu.SMEM),
            pl.BlockSpec((block_rows, D), lambda i: (i, 0)),
            pl.BlockSpec((2, block_rows, D), lambda i: (0, i, 0)),
            pl.BlockSpec((D, H), lambda i: (0, 0)),
            pl.BlockSpec((1, H), lambda i: (0, 0)),
            pl.BlockSpec((H, D), lambda i: (0, 0)),
            pl.BlockSpec((1, D), lambda i: (0, 0)),
            pl.BlockSpec((1, D), lambda i: (0, 0)),
            pl.BlockSpec((1, D), lambda i: (0, 0)),
        ],
        out_specs=pl.BlockSpec((block_rows, D), lambda i: (i, 0)),
        out_shape=jax.ShapeDtypeStruct((N, D), jnp.float32),
    )(
        scal.reshape(1, 1), h, agg,
        w1, b1.reshape(1, H), w2, b2.reshape(1, D),
        gamma.reshape(1, D), beta.reshape(1, D),
    )


def _tc_pool(h, batch, Wp, bp, G):
    N, D = h.shape
    T = Wp.shape[1]

    def body(h_ref, b_ref, wp_ref, bp_ref, o_ref):
        gid = lax.broadcasted_iota(jnp.int32, (G, N), 0)
        pt = jnp.where(b_ref[...] == gid, 1.0, 0.0)
        ssum = jnp.dot(pt, h_ref[...], preferred_element_type=jnp.float32)
        cnt = jnp.dot(pt, jnp.ones((N, 1), jnp.float32),
                      preferred_element_type=jnp.float32)
        pooled = ssum / jnp.maximum(cnt, 1.0)
        o_ref[...] = (
            jnp.dot(pooled, wp_ref[...], preferred_element_type=jnp.float32)
            + bp_ref[...]
        )

    return pl.pallas_call(
        body,
        in_specs=[
            pl.BlockSpec((N, D), lambda: (0, 0)),
            pl.BlockSpec((1, N), lambda: (0, 0)),
            pl.BlockSpec((D, T), lambda: (0, 0)),
            pl.BlockSpec((1, T), lambda: (0, 0)),
        ],
        out_specs=pl.BlockSpec((G, T), lambda: (0, 0)),
        out_shape=jax.ShapeDtypeStruct((G, T), jnp.float32),
    )(h, batch.reshape(1, N), Wp, bp.reshape(1, T))


def kernel(x, edge_index, edge_attr, batch, W_in, b_in, We, eps, W1, b1, W2,
           b2, gamma, beta, Wp, bp):
    N, D = x.shape
    E = edge_index.shape[1]
    L = We.shape[0]
    G = 128
    block_rows = 1000

    chunks_per_worker = -(-E // (_NW * _C))
    e_pad = _NW * chunks_per_worker * _C
    n_table = -(-(N + 1) // (_NS * 8)) * (_NS * 8)

    src = jnp.concatenate([edge_index[0], jnp.zeros((e_pad - E,), jnp.int32)])
    dst = jnp.concatenate(
        [edge_index[1], jnp.full((e_pad - E,), N, jnp.int32)])
    attr_flat = jnp.concatenate(
        [edge_attr.reshape(-1), jnp.zeros(((e_pad - E) * 4,), jnp.float32)])
    ztile = jnp.zeros((n_table // _NS, D), jnp.float32)

    h = _tc_encode(x, W_in, b_in, block_rows)
    for l in range(L):
        agg = _sc_edge_agg(h, src, dst, attr_flat, We[l], ztile,
                           n_table, chunks_per_worker)
        h = _tc_mlp(h, agg, W1[l], b1[l], W2[l], b2[l], gamma[l], beta[l],
                    1.0 + eps[l], relu_out=(l < L - 1), block_rows=block_rows)
    return _tc_pool(h, batch, Wp, bp, G)
